# Initial kernel scaffold; baseline (speedup 1.0000x reference)
import jax
import jax.numpy as jnp
from jax.experimental import pallas as pl


def kernel(x, train_x, train_y, scaler_mean, scaler_std):
    raise NotImplementedError("write your pallas kernel here")


# speedup vs baseline: 2.1392x; 2.1392x over previous
import functools

import jax
import jax.numpy as jnp
from jax.experimental import pallas as pl
from jax.experimental.pallas import tpu as pltpu

_K_NEIGH = 5
_NCLASS = 10
_KB = 1024
_BIG_I = 2 ** 30
_PAD_V = 1.0e6


def _knn_body(x_ref, kt_ref, pid_ref, mr_ref, sr_ref, mc_ref, sc_ref,
              out_ref, bd, bi, *, nsteps):
    i = pl.program_id(0)

    @pl.when(i == 0)
    def _init():
        bd[:] = jnp.full(bd.shape, jnp.inf, jnp.float32)
        bi[:] = jnp.full(bi.shape, _BIG_I, jnp.int32)

    xs = (x_ref[:] - mr_ref[:]) / sr_ref[:]
    q2 = jnp.sum(xs * xs, axis=1, keepdims=True)
    ks = (kt_ref[:] - mc_ref[:]) / sc_ref[:]
    k2 = jnp.sum(ks * ks, axis=0, keepdims=True)
    qk = jnp.dot(xs, ks, preferred_element_type=jnp.float32)
    d2 = (q2 - 2.0 * qk) + k2

    pidb = jnp.broadcast_to(pid_ref[0], d2.shape)

    cand_v, cand_i = [], []
    for _ in range(_K_NEIGH):
        mv = jnp.min(d2, axis=1, keepdims=True)
        mi = jnp.min(jnp.where(d2 == mv, pidb, _BIG_I), axis=1, keepdims=True)
        cand_v.append(mv)
        cand_i.append(mi)
        d2 = jnp.where(pidb == mi, jnp.inf, d2)

    vals = jnp.concatenate([bd[:, 0:_K_NEIGH]] + cand_v, axis=1)
    idxs = jnp.concatenate([bi[:, 0:_K_NEIGH]] + cand_i, axis=1)
    nv, ni = [], []
    for _ in range(_K_NEIGH):
        mv = jnp.min(vals, axis=1, keepdims=True)
        mi = jnp.min(jnp.where(vals == mv, idxs, _BIG_I), axis=1, keepdims=True)
        nv.append(mv)
        ni.append(mi)
        vals = jnp.where(idxs == mi, jnp.inf, vals)
    q = vals.shape[0]
    pad_f = jnp.full((q, bd.shape[1] - _K_NEIGH), jnp.inf, jnp.float32)
    pad_i = jnp.full((q, bi.shape[1] - _K_NEIGH), _BIG_I, jnp.int32)
    bd[:] = jnp.concatenate(nv + [pad_f], axis=1)
    bi[:] = jnp.concatenate(ni + [pad_i], axis=1)

    @pl.when(i == nsteps - 1)
    def _emit():
        lab = jnp.bitwise_and(bi[:, 0:_K_NEIGH], 15)
        cls = jax.lax.broadcasted_iota(jnp.int32, out_ref.shape, 1)
        acc = jnp.zeros(out_ref.shape, jnp.float32)
        for t in range(_K_NEIGH):
            acc = acc + (lab[:, t:t + 1] == cls).astype(jnp.float32)
        out_ref[:] = acc / 5.0


def kernel(x, train_x, train_y, scaler_mean, scaler_std):
    q, d = x.shape
    k = train_x.shape[0]
    nb = pl.cdiv(k, _KB)
    kpad = nb * _KB
    pad = kpad - k

    kt = jnp.pad(train_x.T, ((0, 0), (0, pad)), constant_values=_PAD_V)
    y = jnp.pad(train_y.astype(jnp.int32), (0, pad))
    packed = (jnp.arange(kpad, dtype=jnp.int32) << 4) | y
    packed3 = packed.reshape(nb, 1, _KB)
    mr = scaler_mean.reshape(1, d)
    sr = scaler_std.reshape(1, d)
    mc = scaler_mean.reshape(d, 1)
    sc = scaler_std.reshape(d, 1)

    return pl.pallas_call(
        functools.partial(_knn_body, nsteps=nb),
        grid=(nb,),
        in_specs=[
            pl.BlockSpec((q, d), lambda i: (0, 0)),
            pl.BlockSpec((d, _KB), lambda i: (0, i)),
            pl.BlockSpec((1, 1, _KB), lambda i: (i, 0, 0)),
            pl.BlockSpec((1, d), lambda i: (0, 0)),
            pl.BlockSpec((1, d), lambda i: (0, 0)),
            pl.BlockSpec((d, 1), lambda i: (0, 0)),
            pl.BlockSpec((d, 1), lambda i: (0, 0)),
        ],
        out_specs=pl.BlockSpec((q, _NCLASS), lambda i: (0, 0)),
        out_shape=jax.ShapeDtypeStruct((q, _NCLASS), jnp.float32),
        scratch_shapes=[
            pltpu.VMEM((q, 8), jnp.float32),
            pltpu.VMEM((q, 8), jnp.int32),
        ],
        compiler_params=pltpu.CompilerParams(
            dimension_semantics=("arbitrary",)),
    )(x, kt, packed3, mr, sr, mc, sc)


# speedup vs baseline: 3.5837x; 1.6752x over previous
import functools

import jax
import jax.numpy as jnp
from jax.experimental import pallas as pl
from jax.experimental.pallas import tpu as pltpu

_K_NEIGH = 5
_NCLASS = 10
_KB = 1024
_MERGE_EVERY = 14
_SLOT_W = 8
_CBUF = 128
_BIG_F = 3.0e7
_PAD_V = 1.0e6


def _knn_body(x_ref, kt_ref, pid_ref, mr_ref, sr_ref, mc_ref, sc_ref,
              out_ref, xs_s, q2_s, bd, bi, *, nsteps):
    i = pl.program_id(0)
    nslot = _MERGE_EVERY * _SLOT_W
    q = x_ref.shape[0]

    @pl.when(i == 0)
    def _init():
        xs0 = (x_ref[:] - mr_ref[:]) / sr_ref[:]
        xs_s[:] = xs0
        q2_s[:] = jnp.sum(xs0 * xs0, axis=1, keepdims=True)
        bd[:] = jnp.full(bd.shape, jnp.inf, jnp.float32)
        bi[:] = jnp.full(bi.shape, _BIG_F, jnp.float32)

    xs = xs_s[:]
    q2 = q2_s[:]
    ks = (kt_ref[:] - mc_ref[:]) / sc_ref[:]
    k2 = jnp.sum(ks * ks, axis=0, keepdims=True)
    qk = jnp.dot(xs, ks, preferred_element_type=jnp.float32)
    d2 = (q2 - 2.0 * qk) + k2

    pidb = jnp.broadcast_to(pid_ref[0], d2.shape)

    cv, ci = [], []
    for _ in range(_K_NEIGH):
        mv = jnp.min(d2, axis=1, keepdims=True)
        mi = jnp.min(jnp.where(d2 == mv, pidb, _BIG_F), axis=1, keepdims=True)
        cv.append(mv)
        ci.append(mi)
        d2 = jnp.where(pidb == mi, jnp.inf, d2)

    pad3_f = jnp.full((q, _SLOT_W - _K_NEIGH), jnp.inf, jnp.float32)
    pad3_i = jnp.full((q, _SLOT_W - _K_NEIGH), _BIG_F, jnp.float32)
    cand_f = jnp.concatenate(cv + [pad3_f], axis=1)
    cand_i = jnp.concatenate(ci + [pad3_i], axis=1)
    bd[:] = jnp.concatenate(
        [cand_f, bd[:, 0:nslot - _SLOT_W], bd[:, nslot:_CBUF]], axis=1)
    bi[:] = jnp.concatenate(
        [cand_i, bi[:, 0:nslot - _SLOT_W], bi[:, nslot:_CBUF]], axis=1)

    @pl.when((i % _MERGE_EVERY == _MERGE_EVERY - 1) | (i == nsteps - 1))
    def _merge():
        vals = bd[:]
        idxs = bi[:]
        nv, ni = [], []
        for _ in range(_K_NEIGH):
            mv = jnp.min(vals, axis=1, keepdims=True)
            mi = jnp.min(jnp.where(vals == mv, idxs, _BIG_F),
                         axis=1, keepdims=True)
            nv.append(mv)
            ni.append(mi)
            vals = jnp.where(idxs == mi, jnp.inf, vals)
        reset_f = jnp.full((q, nslot), jnp.inf, jnp.float32)
        reset_i = jnp.full((q, nslot), _BIG_F, jnp.float32)
        tail_f = jnp.full((q, _CBUF - nslot - _K_NEIGH), jnp.inf, jnp.float32)
        tail_i = jnp.full((q, _CBUF - nslot - _K_NEIGH), _BIG_F, jnp.float32)
        bd[:] = jnp.concatenate([reset_f] + nv + [tail_f], axis=1)
        bi[:] = jnp.concatenate([reset_i] + ni + [tail_i], axis=1)

    @pl.when(i == nsteps - 1)
    def _emit():
        packed = bi[:, nslot:nslot + _K_NEIGH].astype(jnp.int32)
        lab = jnp.bitwise_and(packed, 15)
        cls = jax.lax.broadcasted_iota(jnp.int32, out_ref.shape, 1)
        acc = jnp.zeros(out_ref.shape, jnp.float32)
        for t in range(_K_NEIGH):
            acc = acc + (lab[:, t:t + 1] == cls).astype(jnp.float32)
        out_ref[:] = acc / 5.0


def kernel(x, train_x, train_y, scaler_mean, scaler_std):
    q, d = x.shape
    k = train_x.shape[0]
    nb = pl.cdiv(k, _KB)
    kpad = nb * _KB
    pad = kpad - k

    kt = jnp.pad(train_x.T, ((0, 0), (0, pad)), constant_values=_PAD_V)
    y = jnp.pad(train_y.astype(jnp.int32), (0, pad))
    packed = (jnp.arange(kpad, dtype=jnp.int32) << 4) | y
    pid3 = packed.astype(jnp.float32).reshape(nb, 1, _KB)
    mr = scaler_mean.reshape(1, d)
    sr = scaler_std.reshape(1, d)
    mc = scaler_mean.reshape(d, 1)
    sc = scaler_std.reshape(d, 1)

    return pl.pallas_call(
        functools.partial(_knn_body, nsteps=nb),
        grid=(nb,),
        in_specs=[
            pl.BlockSpec((q, d), lambda i: (0, 0)),
            pl.BlockSpec((d, _KB), lambda i: (0, i)),
            pl.BlockSpec((1, 1, _KB), lambda i: (i, 0, 0)),
            pl.BlockSpec((1, d), lambda i: (0, 0)),
            pl.BlockSpec((1, d), lambda i: (0, 0)),
            pl.BlockSpec((d, 1), lambda i: (0, 0)),
            pl.BlockSpec((d, 1), lambda i: (0, 0)),
        ],
        out_specs=pl.BlockSpec((q, _NCLASS), lambda i: (0, 0)),
        out_shape=jax.ShapeDtypeStruct((q, _NCLASS), jnp.float32),
        scratch_shapes=[
            pltpu.VMEM((q, d), jnp.float32),
            pltpu.VMEM((q, 1), jnp.float32),
            pltpu.VMEM((q, _CBUF), jnp.float32),
            pltpu.VMEM((q, _CBUF), jnp.float32),
        ],
        compiler_params=pltpu.CompilerParams(
            dimension_semantics=("arbitrary",)),
    )(x, kt, pid3, mr, sr, mc, sc)


# speedup vs baseline: 5.4301x; 1.5152x over previous
import functools

import jax
import jax.numpy as jnp
from jax.experimental import pallas as pl
from jax.experimental.pallas import tpu as pltpu

_K_NEIGH = 5
_NCLASS = 10
_BIG_F = 3.0e7
_PAD_V = 1.0e6


_KB_F = 2048
_CHUNK = 128
_NCH = _KB_F // _CHUNK


def _scan_body(x_ref, kt_ref, oh_ref, mr_ref, sr_ref, mc_ref, sc_ref,
               probs_ref, c5_ref, xs_s, q2_s, s1, s2, s3, w_s, v_s, *, nb):
    i = pl.program_id(0)
    q = x_ref.shape[0]

    @pl.when(i == 0)
    def _init():
        xs0 = (x_ref[:] - mr_ref[:]) / sr_ref[:]
        xs_s[:] = xs0
        q2_s[:] = jnp.sum(xs0 * xs0, axis=1, keepdims=True)
        s1[:] = jnp.full(s1.shape, jnp.inf, jnp.float32)
        s2[:] = jnp.full(s2.shape, jnp.inf, jnp.float32)
        s3[:] = jnp.full(s3.shape, jnp.inf, jnp.float32)

    xs = xs_s[:]
    q2 = q2_s[:]
    ks = (kt_ref[:] - mc_ref[:]) / sc_ref[:]
    k2 = jnp.sum(ks * ks, axis=0, keepdims=True)

    @pl.when(i < nb)
    def _sweep_a():
        for c in range(_NCH):
            lo = c * _CHUNK
            qk = jnp.dot(xs, ks[:, lo:lo + _CHUNK],
                         preferred_element_type=jnp.float32)
            d2 = (q2 - 2.0 * qk) + k2[:, lo:lo + _CHUNK]
            a = s1[:]
            s1[:] = jnp.minimum(a, d2)
            d2 = jnp.maximum(a, d2)
            b = s2[:]
            s2[:] = jnp.minimum(b, d2)
            d2 = jnp.maximum(b, d2)
            s3[:] = jnp.minimum(s3[:], d2)

    @pl.when(i == nb - 1)
    def _finalize():
        cat = jnp.concatenate([s1[:], s2[:], s3[:]], axis=1)
        ws = []
        for _ in range(_K_NEIGH):
            mv = jnp.min(cat, axis=1, keepdims=True)
            ws.append(mv)
            cat = jnp.where(cat == mv, jnp.inf, cat)
        pad3 = jnp.full((q, 8 - _K_NEIGH), jnp.inf, jnp.float32)
        w_s[:] = jnp.concatenate(ws + [pad3], axis=1)

    @pl.when(i == nb)
    def _vinit():
        v_s[:] = jnp.zeros(v_s.shape, jnp.float32)

    @pl.when(i >= nb)
    def _sweep_b():
        w5 = w_s[:, _K_NEIGH - 1:_K_NEIGH]
        qk = jnp.dot(xs, ks, preferred_element_type=jnp.float32)
        d2 = (q2 - 2.0 * qk) + k2
        le = jnp.where(d2 <= w5, 1.0, 0.0).astype(jnp.float32)
        vb = jnp.dot(le, oh_ref[:], preferred_element_type=jnp.float32)
        v_s[:, 0:_NCLASS] = v_s[:, 0:_NCLASS] + vb

    @pl.when(i == 2 * nb - 1)
    def _emit():
        votes = v_s[:, 0:_NCLASS]
        probs_ref[:] = votes / 5.0
        c5_ref[:] = jnp.sum(votes, axis=1, keepdims=True)


def _fast_call(x, kt, oh, mr, sr, mc, sc, nb):
    q, d = x.shape
    return pl.pallas_call(
        functools.partial(_scan_body, nb=nb),
        grid=(2 * nb,),
        in_specs=[
            pl.BlockSpec((q, d), lambda i: (0, 0)),
            pl.BlockSpec((d, _KB_F), lambda i: (0, jax.lax.rem(i, nb))),
            pl.BlockSpec((_KB_F, _NCLASS),
                         lambda i: (jnp.maximum(i - nb, 0), 0)),
            pl.BlockSpec((1, d), lambda i: (0, 0)),
            pl.BlockSpec((1, d), lambda i: (0, 0)),
            pl.BlockSpec((d, 1), lambda i: (0, 0)),
            pl.BlockSpec((d, 1), lambda i: (0, 0)),
        ],
        out_specs=[
            pl.BlockSpec((q, _NCLASS), lambda i: (0, 0)),
            pl.BlockSpec((q, 1), lambda i: (0, 0)),
        ],
        out_shape=[
            jax.ShapeDtypeStruct((q, _NCLASS), jnp.float32),
            jax.ShapeDtypeStruct((q, 1), jnp.float32),
        ],
        scratch_shapes=[
            pltpu.VMEM((q, d), jnp.float32),
            pltpu.VMEM((q, 1), jnp.float32),
            pltpu.VMEM((q, _CHUNK), jnp.float32),
            pltpu.VMEM((q, _CHUNK), jnp.float32),
            pltpu.VMEM((q, _CHUNK), jnp.float32),
            pltpu.VMEM((q, 8), jnp.float32),
            pltpu.VMEM((q, _NCLASS), jnp.float32),
        ],
        compiler_params=pltpu.CompilerParams(
            dimension_semantics=("arbitrary",)),
    )(x, kt, oh, mr, sr, mc, sc)


_KB = 1024
_MERGE_EVERY = 14
_SLOT_W = 8
_CBUF = 128


def _exact_body(x_ref, kt_ref, pid_ref, mr_ref, sr_ref, mc_ref, sc_ref,
                out_ref, xs_s, q2_s, bd, bi, *, nsteps):
    i = pl.program_id(0)
    nslot = _MERGE_EVERY * _SLOT_W
    q = x_ref.shape[0]

    @pl.when(i == 0)
    def _init():
        xs0 = (x_ref[:] - mr_ref[:]) / sr_ref[:]
        xs_s[:] = xs0
        q2_s[:] = jnp.sum(xs0 * xs0, axis=1, keepdims=True)
        bd[:] = jnp.full(bd.shape, jnp.inf, jnp.float32)
        bi[:] = jnp.full(bi.shape, _BIG_F, jnp.float32)

    xs = xs_s[:]
    q2 = q2_s[:]
    ks = (kt_ref[:] - mc_ref[:]) / sc_ref[:]
    k2 = jnp.sum(ks * ks, axis=0, keepdims=True)
    qk = jnp.dot(xs, ks, preferred_element_type=jnp.float32)
    d2 = (q2 - 2.0 * qk) + k2

    pidb = jnp.broadcast_to(pid_ref[0], d2.shape)

    cv, ci = [], []
    for t in range(_K_NEIGH):
        mv = jnp.min(d2, axis=1, keepdims=True)
        mi = jnp.min(jnp.where(d2 == mv, pidb, _BIG_F), axis=1, keepdims=True)
        cv.append(mv)
        ci.append(mi)
        if t < _K_NEIGH - 1:
            d2 = jnp.where(pidb == mi, jnp.inf, d2)

    pad3_f = jnp.full((q, _SLOT_W - _K_NEIGH), jnp.inf, jnp.float32)
    pad3_i = jnp.full((q, _SLOT_W - _K_NEIGH), _BIG_F, jnp.float32)
    cand_f = jnp.concatenate(cv + [pad3_f], axis=1)
    cand_i = jnp.concatenate(ci + [pad3_i], axis=1)
    bd[:] = jnp.concatenate(
        [cand_f, bd[:, 0:nslot - _SLOT_W], bd[:, nslot:_CBUF]], axis=1)
    bi[:] = jnp.concatenate(
        [cand_i, bi[:, 0:nslot - _SLOT_W], bi[:, nslot:_CBUF]], axis=1)

    @pl.when((i % _MERGE_EVERY == _MERGE_EVERY - 1) | (i == nsteps - 1))
    def _merge():
        vals = bd[:]
        idxs = bi[:]
        nv, ni = [], []
        for _ in range(_K_NEIGH):
            mv = jnp.min(vals, axis=1, keepdims=True)
            mi = jnp.min(jnp.where(vals == mv, idxs, _BIG_F),
                         axis=1, keepdims=True)
            nv.append(mv)
            ni.append(mi)
            vals = jnp.where(idxs == mi, jnp.inf, vals)
        reset_f = jnp.full((q, nslot), jnp.inf, jnp.float32)
        reset_i = jnp.full((q, nslot), _BIG_F, jnp.float32)
        tail_f = jnp.full((q, _CBUF - nslot - _K_NEIGH), jnp.inf, jnp.float32)
        tail_i = jnp.full((q, _CBUF - nslot - _K_NEIGH), _BIG_F, jnp.float32)
        bd[:] = jnp.concatenate([reset_f] + nv + [tail_f], axis=1)
        bi[:] = jnp.concatenate([reset_i] + ni + [tail_i], axis=1)

    @pl.when(i == nsteps - 1)
    def _emit():
        packed = bi[:, nslot:nslot + _K_NEIGH].astype(jnp.int32)
        lab = jnp.bitwise_and(packed, 15)
        cls = jax.lax.broadcasted_iota(jnp.int32, out_ref.shape, 1)
        acc = jnp.zeros(out_ref.shape, jnp.float32)
        for t in range(_K_NEIGH):
            acc = acc + (lab[:, t:t + 1] == cls).astype(jnp.float32)
        out_ref[:] = acc / 5.0


def _exact_call(x, kt, pid3, mr, sr, mc, sc, nb):
    q, d = x.shape
    return pl.pallas_call(
        functools.partial(_exact_body, nsteps=nb),
        grid=(nb,),
        in_specs=[
            pl.BlockSpec((q, d), lambda i: (0, 0)),
            pl.BlockSpec((d, _KB), lambda i: (0, i)),
            pl.BlockSpec((1, 1, _KB), lambda i: (i, 0, 0)),
            pl.BlockSpec((1, d), lambda i: (0, 0)),
            pl.BlockSpec((1, d), lambda i: (0, 0)),
            pl.BlockSpec((d, 1), lambda i: (0, 0)),
            pl.BlockSpec((d, 1), lambda i: (0, 0)),
        ],
        out_specs=pl.BlockSpec((q, _NCLASS), lambda i: (0, 0)),
        out_shape=jax.ShapeDtypeStruct((q, _NCLASS), jnp.float32),
        scratch_shapes=[
            pltpu.VMEM((q, d), jnp.float32),
            pltpu.VMEM((q, 1), jnp.float32),
            pltpu.VMEM((q, _CBUF), jnp.float32),
            pltpu.VMEM((q, _CBUF), jnp.float32),
        ],
        compiler_params=pltpu.CompilerParams(
            dimension_semantics=("arbitrary",)),
    )(x, kt, pid3, mr, sr, mc, sc)


def kernel(x, train_x, train_y, scaler_mean, scaler_std):
    q, d = x.shape
    k = train_x.shape[0]
    nb_f = pl.cdiv(k, _KB_F)
    kpad = nb_f * _KB_F
    pad = kpad - k

    kt = jnp.pad(train_x.T, ((0, 0), (0, pad)), constant_values=_PAD_V)
    y = train_y.astype(jnp.int32)
    ypad = jnp.pad(y, (0, pad), constant_values=-1)
    oh = (ypad[:, None] == jnp.arange(_NCLASS, dtype=jnp.int32)[None, :])
    oh = oh.astype(jnp.float32)
    mr = scaler_mean.reshape(1, d)
    sr = scaler_std.reshape(1, d)
    mc = scaler_mean.reshape(d, 1)
    sc = scaler_std.reshape(d, 1)

    probs_fast, c5 = _fast_call(x, kt, oh, mr, sr, mc, sc, nb_f)

    nb_e = kpad // _KB
    packed = (jnp.arange(kpad, dtype=jnp.int32) << 4) | jnp.maximum(ypad, 0)
    pid3 = packed.astype(jnp.float32).reshape(nb_e, 1, _KB)

    tie = jnp.any(c5 != 5.0)
    return jax.lax.cond(
        tie,
        lambda: _exact_call(x, kt, pid3, mr, sr, mc, sc, nb_e),
        lambda: probs_fast,
    )
